# Initial kernel scaffold; baseline (speedup 1.0000x reference)
import jax
import jax.numpy as jnp
from jax.experimental import pallas as pl


def kernel(x, Wq, bq, Wk, bk, Wv, bv, Wo, bo):
    raise NotImplementedError("write your pallas kernel here")


# speedup vs baseline: 29.9539x; 29.9539x over previous
import math

import jax
import jax.numpy as jnp
import numpy as np
from jax.experimental import pallas as pl

_DIM = 1024
_KQ = 64
_VAL = 64
_H = 16
_K = 32
_B = 2
_N = 2048

_MR = 512
_QB = 512

_I32_MIN = np.int32(-(2 ** 31))
_I16_MIN = np.int16(-(2 ** 15))


def _matmul_bias_kernel(x_ref, w_ref, b_ref, o_ref):
    o_ref[...] = (
        jax.lax.dot_general(
            x_ref[...], w_ref[...], (((1,), (0,)), ((), ())),
            preferred_element_type=jnp.float32)
        + b_ref[...])


def _matmul_bias(x, w, b, br, bc):
    m, kdim = x.shape
    _, n = w.shape
    return pl.pallas_call(
        _matmul_bias_kernel,
        grid=(m // br, n // bc),
        in_specs=[
            pl.BlockSpec((br, kdim), lambda i, j: (i, 0)),
            pl.BlockSpec((kdim, bc), lambda i, j: (0, j)),
            pl.BlockSpec((1, bc), lambda i, j: (0, j)),
        ],
        out_specs=pl.BlockSpec((br, bc), lambda i, j: (i, j)),
        out_shape=jax.ShapeDtypeStruct((m, n), jnp.float32),
    )(x, w, b.reshape(1, n))


def _count16(mask_i16_01):
    x = mask_i16_01
    while x.shape[1] > 256:
        half = x.shape[1] // 2
        x = x[:, :half] + x[:, half:]
    return jnp.sum(x.astype(jnp.int32), axis=1, keepdims=True)


def _attn_kernel(q_ref, k_ref, v_ref, o_ref):
    q = q_ref[0, 0]
    k = k_ref[0, 0]
    v = v_ref[0, 0]
    rows = q.shape[0]
    s = jax.lax.dot_general(
        q, k, (((1,), (1,)), ((), ())),
        preferred_element_type=jnp.float32) * (1.0 / math.sqrt(_KQ))

    bits = jax.lax.bitcast_convert_type(s, jnp.int32)
    key = jnp.where(bits < 0,
                    jnp.bitwise_xor(jnp.bitwise_not(bits), _I32_MIN),
                    bits)
    khi = jax.lax.shift_right_arithmetic(key, np.int32(16)).astype(jnp.int16)
    klo = jnp.bitwise_xor(key.astype(jnp.int16), _I16_MIN)

    one16 = np.int16(1)
    zero16 = np.int16(0)

    def hi_body(i, tu):
        bit = np.int32(15) - i
        cand = jnp.bitwise_or(tu, jnp.left_shift(np.int32(1), bit))
        cand_s = jnp.bitwise_xor(cand, np.int32(0x8000)).astype(jnp.int16)
        cnt = _count16(jnp.where(khi >= cand_s, one16, zero16))
        return jnp.where(cnt >= _K, cand, tu)

    tu_hi = jax.lax.fori_loop(0, 16, hi_body,
                              jnp.zeros((rows, 1), jnp.int32))
    ts_hi = jnp.bitwise_xor(tu_hi, np.int32(0x8000)).astype(jnp.int16)

    eqhi = khi == ts_hi
    c_gt_hi = _count16(jnp.where(khi > ts_hi, one16, zero16))
    kth_lo = _K - c_gt_hi

    def lo_body(i, tu):
        bit = np.int32(15) - i
        cand = jnp.bitwise_or(tu, jnp.left_shift(np.int32(1), bit))
        cand_s = jnp.bitwise_xor(cand, np.int32(0x8000)).astype(jnp.int16)
        cnt = _count16(jnp.where(eqhi & (klo >= cand_s), one16, zero16))
        return jnp.where(cnt >= kth_lo, cand, tu)

    tu_lo = jax.lax.fori_loop(0, 16, lo_body,
                              jnp.zeros((rows, 1), jnp.int32))
    ts_lo = jnp.bitwise_xor(tu_lo, np.int32(0x8000)).astype(jnp.int16)

    gt = (khi > ts_hi) | (eqhi & (klo > ts_lo))
    eq = eqhi & (klo == ts_lo)
    cnt_gt = _count16(jnp.where(gt, one16, zero16))
    cnt_eq = _count16(jnp.where(eq, one16, zero16))
    has_tie = jnp.any(cnt_gt + cnt_eq != _K)

    need = _K - cnt_gt
    idx = jax.lax.broadcasted_iota(jnp.int32, s.shape, 1)

    def _tie_cutoff(_):
        def ibody(i, it):
            bit = np.int32(11) - i
            cand = jnp.bitwise_or(it, jnp.left_shift(np.int32(1), bit))
            c = jnp.sum((eq & (idx < cand)).astype(jnp.int32),
                        axis=1, keepdims=True)
            return jnp.where(c <= need, cand, it)
        return jax.lax.fori_loop(0, 12, ibody,
                                 jnp.zeros((rows, 1), jnp.int32))

    it = jax.lax.cond(has_tie, _tie_cutoff,
                      lambda _: jnp.full((rows, 1), np.int32(_N)),
                      operand=None)

    mask = gt | (eq & (idx < it))
    rowmax = jnp.max(s, axis=1, keepdims=True)
    e = jnp.where(mask, jnp.exp(s - rowmax), 0.0)
    p = e / jnp.sum(e, axis=1, keepdims=True)
    o_ref[0, 0] = jax.lax.dot_general(
        p, v, (((1,), (0,)), ((), ())),
        preferred_element_type=jnp.float32)


def _attention(q, k, v):
    return pl.pallas_call(
        _attn_kernel,
        grid=(_B, _H, _N // _QB),
        in_specs=[
            pl.BlockSpec((1, 1, _QB, _KQ), lambda b, h, i: (b, h, i, 0)),
            pl.BlockSpec((1, 1, _N, _KQ), lambda b, h, i: (b, h, 0, 0)),
            pl.BlockSpec((1, 1, _N, _VAL), lambda b, h, i: (b, h, 0, 0)),
        ],
        out_specs=pl.BlockSpec((1, 1, _QB, _VAL), lambda b, h, i: (b, h, i, 0)),
        out_shape=jax.ShapeDtypeStruct((_B, _H, _N, _VAL), jnp.float32),
    )(q, k, v)


def kernel(x, Wq, bq, Wk, bk, Wv, bv, Wo, bo):
    b, n, dim = x.shape
    w = jnp.concatenate([Wq, Wk, Wv], axis=1)
    bias = jnp.concatenate([bq, bk, bv], axis=0)
    qkv = _matmul_bias(x.reshape(b * n, dim), w, bias, _MR, 1024)
    qkv = qkv.reshape(b, n, 3, _H, _KQ).transpose(2, 0, 3, 1, 4)
    attn = _attention(qkv[0], qkv[1], qkv[2])
    attn = attn.transpose(0, 2, 1, 3).reshape(b * n, _H * _VAL)
    out = _matmul_bias(attn, Wo, bo, _MR, 1024)
    return out.reshape(b, n, dim)


# speedup vs baseline: 31.7343x; 1.0594x over previous
import math

import jax
import jax.numpy as jnp
import numpy as np
from jax.experimental import pallas as pl

_DIM = 1024
_KQ = 64
_VAL = 64
_H = 16
_K = 32
_B = 2
_N = 2048

_MR = 512
_QB = 1024

_I32_MIN = np.int32(-(2 ** 31))
_I16_MIN = np.int16(-(2 ** 15))


def _matmul_bias_kernel(x_ref, w_ref, b_ref, o_ref):
    o_ref[...] = (
        jax.lax.dot_general(
            x_ref[...], w_ref[...], (((1,), (0,)), ((), ())),
            preferred_element_type=jnp.float32)
        + b_ref[...])


def _matmul_bias(x, w, b, br, bc):
    m, kdim = x.shape
    _, n = w.shape
    return pl.pallas_call(
        _matmul_bias_kernel,
        grid=(m // br, n // bc),
        in_specs=[
            pl.BlockSpec((br, kdim), lambda i, j: (i, 0)),
            pl.BlockSpec((kdim, bc), lambda i, j: (0, j)),
            pl.BlockSpec((1, bc), lambda i, j: (0, j)),
        ],
        out_specs=pl.BlockSpec((br, bc), lambda i, j: (i, j)),
        out_shape=jax.ShapeDtypeStruct((m, n), jnp.float32),
    )(x, w, b.reshape(1, n))


def _count16(mask_i16_01):
    x = mask_i16_01
    while x.shape[1] > 256:
        half = x.shape[1] // 2
        x = x[:, :half] + x[:, half:]
    return jnp.sum(x.astype(jnp.int32), axis=1, keepdims=True)


def _attn_kernel(q_ref, k_ref, v_ref, o_ref):
    q = q_ref[0, 0]
    k = k_ref[0, 0]
    v = v_ref[0, 0]
    rows = q.shape[0]
    s = jax.lax.dot_general(
        q, k, (((1,), (1,)), ((), ())),
        preferred_element_type=jnp.float32) * (1.0 / math.sqrt(_KQ))

    bits = jax.lax.bitcast_convert_type(s, jnp.int32)
    key = jnp.where(bits < 0,
                    jnp.bitwise_xor(jnp.bitwise_not(bits), _I32_MIN),
                    bits)
    khi = jax.lax.shift_right_arithmetic(key, np.int32(16)).astype(jnp.int16)
    klo = jnp.bitwise_xor(key.astype(jnp.int16), _I16_MIN)

    one16 = np.int16(1)
    zero16 = np.int16(0)

    def hi_body(i, tu):
        bit = np.int32(15) - i
        cand = jnp.bitwise_or(tu, jnp.left_shift(np.int32(1), bit))
        cand_s = jnp.bitwise_xor(cand, np.int32(0x8000)).astype(jnp.int16)
        cnt = _count16(jnp.where(khi >= cand_s, one16, zero16))
        return jnp.where(cnt >= _K, cand, tu)

    tu_hi = jax.lax.fori_loop(0, 16, hi_body,
                              jnp.zeros((rows, 1), jnp.int32))
    ts_hi = jnp.bitwise_xor(tu_hi, np.int32(0x8000)).astype(jnp.int16)

    eqhi = khi == ts_hi
    c_gt_hi = _count16(jnp.where(khi > ts_hi, one16, zero16))
    kth_lo = _K - c_gt_hi

    def lo_body(i, tu):
        bit = np.int32(15) - i
        cand = jnp.bitwise_or(tu, jnp.left_shift(np.int32(1), bit))
        cand_s = jnp.bitwise_xor(cand, np.int32(0x8000)).astype(jnp.int16)
        cnt = _count16(jnp.where(eqhi & (klo >= cand_s), one16, zero16))
        return jnp.where(cnt >= kth_lo, cand, tu)

    tu_lo = jax.lax.fori_loop(0, 16, lo_body,
                              jnp.zeros((rows, 1), jnp.int32))
    ts_lo = jnp.bitwise_xor(tu_lo, np.int32(0x8000)).astype(jnp.int16)

    gt = (khi > ts_hi) | (eqhi & (klo > ts_lo))
    eq = eqhi & (klo == ts_lo)
    cnt_gt = _count16(jnp.where(gt, one16, zero16))
    cnt_eq = _count16(jnp.where(eq, one16, zero16))
    has_tie = jnp.any(cnt_gt + cnt_eq != _K)

    need = _K - cnt_gt
    idx = jax.lax.broadcasted_iota(jnp.int32, s.shape, 1)

    def _tie_cutoff(_):
        def ibody(i, it):
            bit = np.int32(11) - i
            cand = jnp.bitwise_or(it, jnp.left_shift(np.int32(1), bit))
            c = jnp.sum((eq & (idx < cand)).astype(jnp.int32),
                        axis=1, keepdims=True)
            return jnp.where(c <= need, cand, it)
        return jax.lax.fori_loop(0, 12, ibody,
                                 jnp.zeros((rows, 1), jnp.int32))

    it = jax.lax.cond(has_tie, _tie_cutoff,
                      lambda _: jnp.full((rows, 1), np.int32(_N)),
                      operand=None)

    mask = gt | (eq & (idx < it))
    rowmax = jnp.max(s, axis=1, keepdims=True)
    e = jnp.where(mask, jnp.exp(s - rowmax), 0.0)
    p = e / jnp.sum(e, axis=1, keepdims=True)
    o_ref[0, 0] = jax.lax.dot_general(
        p, v, (((1,), (0,)), ((), ())),
        preferred_element_type=jnp.float32)


def _attention(q, k, v):
    return pl.pallas_call(
        _attn_kernel,
        grid=(_B, _H, _N // _QB),
        in_specs=[
            pl.BlockSpec((1, 1, _QB, _KQ), lambda b, h, i: (b, h, i, 0)),
            pl.BlockSpec((1, 1, _N, _KQ), lambda b, h, i: (b, h, 0, 0)),
            pl.BlockSpec((1, 1, _N, _VAL), lambda b, h, i: (b, h, 0, 0)),
        ],
        out_specs=pl.BlockSpec((1, 1, _QB, _VAL), lambda b, h, i: (b, h, i, 0)),
        out_shape=jax.ShapeDtypeStruct((_B, _H, _N, _VAL), jnp.float32),
    )(q, k, v)


def kernel(x, Wq, bq, Wk, bk, Wv, bv, Wo, bo):
    b, n, dim = x.shape
    w = jnp.concatenate([Wq, Wk, Wv], axis=1)
    bias = jnp.concatenate([bq, bk, bv], axis=0)
    qkv = _matmul_bias(x.reshape(b * n, dim), w, bias, _MR, 1024)
    qkv = qkv.reshape(b, n, 3, _H, _KQ).transpose(2, 0, 3, 1, 4)
    attn = _attention(qkv[0], qkv[1], qkv[2])
    attn = attn.transpose(0, 2, 1, 3).reshape(b * n, _H * _VAL)
    out = _matmul_bias(attn, Wo, bo, _MR, 1024)
    return out.reshape(b, n, dim)
